# baseline (speedup 1.0000x reference)
import functools
import math

import jax
import jax.numpy as jnp
from jax import lax
from jax.experimental import pallas as pl
from jax.experimental.pallas import tpu as pltpu
from jax.experimental.pallas import tpu_sc as plsc

D_MODEL = 64
SCALE = math.sqrt(D_MODEL)
VOCAB = 1000000
PAD_D = 128

_info = plsc.get_sparse_core_info()
NC, NS, L = _info.num_cores, _info.num_subcores, _info.num_lanes
NW = NC * NS

SEQ = 200
BATCH = 4096
CHUNK = 128
JBLK = BATCH // CHUNK
N_CHUNKS = SEQ * JBLK
ROWS_PER_W = N_CHUNKS // NW
NBUF = 4
N_BLOCKS = ROWS_PER_W // NBUF


N_RPK = VOCAB // CHUNK


@functools.partial(
    pl.kernel,
    out_type=jax.ShapeDtypeStruct((VOCAB, PAD_D), jnp.float32),
    mesh=plsc.VectorSubcoreMesh(core_axis_name="c", subcore_axis_name="s"),
    scratch_types=[
        pltpu.VMEM((2, D_MODEL, CHUNK), jnp.float32),
        pltpu.VMEM((2, CHUNK, PAD_D), jnp.float32),
        pltpu.SemaphoreType.DMA((2,)),
    ],
    compiler_params=pltpu.CompilerParams(needs_layout_passes=False),
)
def _repack_sc(src_hbm, tail_hbm, dst_hbm, sbuf, obuf, osem):
    wid = lax.axis_index("s") * NC + lax.axis_index("c")
    nb = jnp.where(wid < N_RPK % NW, N_RPK // NW + 1, N_RPK // NW)

    lanes = jnp.arange(L, dtype=jnp.int32)
    masks = {k: (lanes & k) != 0 for k in (8, 4, 2, 1)}
    perm_sub = {k: (lanes - k) % L for k in (8, 4, 2, 1)}
    perm_add = {k: (lanes + k) % L for k in (8, 4, 2, 1)}

    @pl.when(wid == NW - 1)
    def _tail():
        pltpu.sync_copy(tail_hbm, obuf.at[0])
        pltpu.sync_copy(
            obuf.at[0, pl.ds(0, D_MODEL), :],
            dst_hbm.at[pl.ds(N_RPK * CHUNK, D_MODEL)],
        )

    def body(i, carry):
        p = lax.rem(i, 2)
        c0 = pl.multiple_of((wid + i * NW) * CHUNK, CHUNK)

        def _drain():
            pltpu.make_async_copy(
                obuf.at[p], dst_hbm.at[pl.ds(0, CHUNK)], osem.at[p]
            ).wait()

        pl.when(i > 1)(_drain)
        pltpu.sync_copy(src_hbm.at[:, pl.ds(c0, CHUNK)], sbuf.at[p])

        def tr_blocks(g, c2):
            c16 = g * L
            for h in range(D_MODEL // L):
                d0 = h * L
                m = [sbuf[p, d0 + i2, pl.ds(c16, L)] for i2 in range(L)]
                for k in (8, 4, 2, 1):
                    for i2 in range(L):
                        if i2 & k == 0:
                            j2 = i2 | k
                            rr = jnp.take_along_axis(m[j2], perm_sub[k], axis=0)
                            ra = jnp.take_along_axis(m[i2], perm_add[k], axis=0)
                            m[i2] = jnp.where(masks[k], rr, m[i2])
                            m[j2] = jnp.where(masks[k], m[j2], ra)
                for j2 in range(L):
                    obuf[p, c16 + j2, pl.ds(d0, L)] = m[j2]
            return c2

        lax.fori_loop(0, CHUNK // L, tr_blocks, 0)
        pltpu.async_copy(obuf.at[p], dst_hbm.at[pl.ds(c0, CHUNK)], osem.at[p])
        return carry

    lax.fori_loop(0, nb, body, 0)
    for p in range(2):
        pltpu.make_async_copy(
            obuf.at[p], dst_hbm.at[pl.ds(0, CHUNK)], osem.at[p]
        ).wait()


@functools.partial(
    pl.kernel,
    out_type=jax.ShapeDtypeStruct((SEQ, D_MODEL, BATCH), jnp.float32),
    mesh=plsc.VectorSubcoreMesh(core_axis_name="c", subcore_axis_name="s"),
    scratch_types=[
        pltpu.VMEM((2, NBUF, CHUNK), jnp.int32),
        pltpu.VMEM((NBUF, CHUNK, PAD_D), jnp.float32),
        pltpu.VMEM((2, D_MODEL, CHUNK), jnp.float32),
        pltpu.SemaphoreType.DMA((2,)),
        pltpu.SemaphoreType.DMA((NBUF,)),
        pltpu.SemaphoreType.DMA((2,)),
    ],
    compiler_params=pltpu.CompilerParams(needs_layout_passes=False),
)
def _embed_sc(lut_hbm, idx_hbm, out_hbm, idx_v, rows_v, plane_v, isem, gsem, osem):
    wid = lax.axis_index("s") * NC + lax.axis_index("c")
    wrow0 = wid * ROWS_PER_W

    lanes = jnp.arange(L, dtype=jnp.int32)
    masks = {k: (lanes & k) != 0 for k in (8, 4, 2, 1)}
    perm_sub = {k: (lanes - k) % L for k in (8, 4, 2, 1)}
    perm_add = {k: (lanes + k) % L for k in (8, 4, 2, 1)}

    pltpu.async_copy(idx_hbm.at[pl.ds(wrow0, NBUF)], idx_v.at[0], isem.at[0])

    def block_body(t, carry):
        p = lax.rem(t, 2)
        pn = lax.rem(t + 1, 2)
        tn = lax.min(t + 1, N_BLOCKS - 1)
        pltpu.async_copy(
            idx_hbm.at[pl.ds(wrow0 + tn * NBUF, NBUF)], idx_v.at[pn],
            isem.at[pn],
        )
        pltpu.make_async_copy(
            idx_hbm.at[pl.ds(0, NBUF)], idx_v.at[p], isem.at[p]
        ).wait()
        chunk0 = t * NBUF
        for b in range(NBUF):
            pltpu.async_copy(
                lut_hbm.at[idx_v.at[p, b]], rows_v.at[b], gsem.at[b]
            )
        for b in range(NBUF):
            pb = b % 2
            pltpu.make_async_copy(
                lut_hbm.at[idx_v.at[p, b]], rows_v.at[b], gsem.at[b]
            ).wait()

            def _drain_plane():
                pltpu.make_async_copy(
                    plane_v.at[pb], out_hbm.at[0, :, pl.ds(0, CHUNK)],
                    osem.at[pb],
                ).wait()

            if b >= 2:
                _drain_plane()
            else:
                pl.when(t > 0)(_drain_plane)

            def tr_blocks(g, c2):
                b16 = g * L
                for h in range(D_MODEL // L):
                    d0 = h * L
                    m = [
                        rows_v[b, b16 + i, pl.ds(d0, L)] * SCALE
                        for i in range(L)
                    ]
                    for k in (8, 4, 2, 1):
                        for i in range(L):
                            if i & k == 0:
                                j = i | k
                                rr = jnp.take_along_axis(
                                    m[j], perm_sub[k], axis=0
                                )
                                ra = jnp.take_along_axis(
                                    m[i], perm_add[k], axis=0
                                )
                                m[i] = jnp.where(masks[k], rr, m[i])
                                m[j] = jnp.where(masks[k], m[j], ra)
                    for j in range(L):
                        plane_v[pb, d0 + j, pl.ds(b16, L)] = m[j]
                return c2

            lax.fori_loop(0, CHUNK // L, tr_blocks, 0)

            k_flat = wrow0 + chunk0 + b
            s = k_flat // JBLK
            b0 = pl.multiple_of((k_flat % JBLK) * CHUNK, CHUNK)
            pltpu.async_copy(
                plane_v.at[pb], out_hbm.at[s, :, pl.ds(b0, CHUNK)],
                osem.at[pb],
            )
        return carry

    lax.fori_loop(0, N_BLOCKS, block_body, 0)
    for pb in range(2):
        pltpu.make_async_copy(
            plane_v.at[pb], out_hbm.at[0, :, pl.ds(0, CHUNK)], osem.at[pb]
        ).wait()
    pltpu.make_async_copy(
        idx_hbm.at[pl.ds(0, NBUF)], idx_v.at[N_BLOCKS % 2],
        isem.at[N_BLOCKS % 2],
    ).wait()


def kernel(x, lut):
    lut_t = jnp.swapaxes(lut, 0, 1)
    tail = jnp.pad(
        lut[N_RPK * CHUNK :, :], ((0, CHUNK - D_MODEL), (0, PAD_D - D_MODEL))
    )
    table = _repack_sc(lut_t, tail)
    idx = jnp.swapaxes(x, 0, 1).astype(jnp.int32).reshape(N_CHUNKS, CHUNK)
    o_t = _embed_sc(table, idx)
    return o_t.transpose(2, 0, 1)


# speedup vs baseline: 1.3393x; 1.3393x over previous
import functools
import math

import jax
import jax.numpy as jnp
from jax import lax
from jax.experimental import pallas as pl
from jax.experimental.pallas import tpu as pltpu
from jax.experimental.pallas import tpu_sc as plsc

D_MODEL = 64
SCALE = math.sqrt(D_MODEL)
VOCAB = 1000000
PAD_D = 128

_info = plsc.get_sparse_core_info()
NC, NS, L = _info.num_cores, _info.num_subcores, _info.num_lanes
NW = NC * NS

SEQ = 200
BATCH = 4096
CHUNK = 128
JBLK = BATCH // CHUNK
N_CHUNKS = SEQ * JBLK
ROWS_PER_W = N_CHUNKS // NW
NBUF = 4
N_BLOCKS = ROWS_PER_W // NBUF


N_RPK = VOCAB // CHUNK


@functools.partial(
    pl.kernel,
    out_type=jax.ShapeDtypeStruct((VOCAB, PAD_D), jnp.float32),
    mesh=plsc.VectorSubcoreMesh(core_axis_name="c", subcore_axis_name="s"),
    scratch_types=[
        pltpu.VMEM((2, D_MODEL, CHUNK), jnp.float32),
        pltpu.VMEM((2, CHUNK, PAD_D), jnp.float32),
        pltpu.SemaphoreType.DMA((2,)),
        pltpu.SemaphoreType.DMA((2,)),
    ],
    compiler_params=pltpu.CompilerParams(needs_layout_passes=False),
)
def _repack_sc(src_hbm, tail_hbm, dst_hbm, sbuf, obuf, osem, rsem):
    wid = lax.axis_index("s") * NC + lax.axis_index("c")
    nb = jnp.where(wid < N_RPK % NW, N_RPK // NW + 1, N_RPK // NW)

    lanes = jnp.arange(L, dtype=jnp.int32)
    masks = {k: (lanes & k) != 0 for k in (8, 4, 2, 1)}
    perm_sub = {k: (lanes - k) % L for k in (8, 4, 2, 1)}
    perm_add = {k: (lanes + k) % L for k in (8, 4, 2, 1)}

    @pl.when(wid == NW - 1)
    def _tail():
        pltpu.sync_copy(tail_hbm, obuf.at[0])
        pltpu.sync_copy(
            obuf.at[0, pl.ds(0, D_MODEL), :],
            dst_hbm.at[pl.ds(N_RPK * CHUNK, D_MODEL)],
        )

    c_first = pl.multiple_of(wid * CHUNK, CHUNK)
    pltpu.async_copy(
        src_hbm.at[:, pl.ds(c_first, CHUNK)], sbuf.at[0], rsem.at[0]
    )

    def body(i, carry):
        p = lax.rem(i, 2)
        pn = lax.rem(i + 1, 2)
        c0 = pl.multiple_of((wid + i * NW) * CHUNK, CHUNK)
        cn = pl.multiple_of(
            (wid + lax.min(i + 1, nb - 1) * NW) * CHUNK, CHUNK
        )
        pltpu.async_copy(
            src_hbm.at[:, pl.ds(cn, CHUNK)], sbuf.at[pn], rsem.at[pn]
        )
        pltpu.make_async_copy(
            src_hbm.at[:, pl.ds(0, CHUNK)], sbuf.at[p], rsem.at[p]
        ).wait()

        def _drain():
            pltpu.make_async_copy(
                obuf.at[p], dst_hbm.at[pl.ds(0, CHUNK)], osem.at[p]
            ).wait()

        pl.when(i > 1)(_drain)

        def tr_blocks(g, c2):
            c16 = g * L
            for h in range(D_MODEL // L):
                d0 = h * L
                m = [sbuf[p, d0 + i2, pl.ds(c16, L)] for i2 in range(L)]
                for k in (8, 4, 2, 1):
                    for i2 in range(L):
                        if i2 & k == 0:
                            j2 = i2 | k
                            rr = jnp.take_along_axis(m[j2], perm_sub[k], axis=0)
                            ra = jnp.take_along_axis(m[i2], perm_add[k], axis=0)
                            m[i2] = jnp.where(masks[k], rr, m[i2])
                            m[j2] = jnp.where(masks[k], m[j2], ra)
                for j2 in range(L):
                    obuf[p, c16 + j2, pl.ds(d0, L)] = m[j2]
            return c2

        lax.fori_loop(0, CHUNK // L, tr_blocks, 0)
        pltpu.async_copy(obuf.at[p], dst_hbm.at[pl.ds(c0, CHUNK)], osem.at[p])
        return carry

    lax.fori_loop(0, nb, body, 0)
    pltpu.make_async_copy(
        src_hbm.at[:, pl.ds(0, CHUNK)], sbuf.at[lax.rem(nb, 2)],
        rsem.at[lax.rem(nb, 2)],
    ).wait()
    for p in range(2):
        pltpu.make_async_copy(
            obuf.at[p], dst_hbm.at[pl.ds(0, CHUNK)], osem.at[p]
        ).wait()


@functools.partial(
    pl.kernel,
    out_type=jax.ShapeDtypeStruct((SEQ, D_MODEL, BATCH), jnp.float32),
    mesh=plsc.VectorSubcoreMesh(core_axis_name="c", subcore_axis_name="s"),
    scratch_types=[
        pltpu.VMEM((2, NBUF, CHUNK), jnp.int32),
        pltpu.VMEM((NBUF, CHUNK, PAD_D), jnp.float32),
        pltpu.VMEM((2, D_MODEL, CHUNK), jnp.float32),
        pltpu.SemaphoreType.DMA((2,)),
        pltpu.SemaphoreType.DMA((NBUF,)),
        pltpu.SemaphoreType.DMA((2,)),
    ],
    compiler_params=pltpu.CompilerParams(needs_layout_passes=False),
)
def _embed_sc(lut_hbm, idx_hbm, out_hbm, idx_v, rows_v, plane_v, isem, gsem, osem):
    wid = lax.axis_index("s") * NC + lax.axis_index("c")
    wrow0 = wid * ROWS_PER_W

    lanes = jnp.arange(L, dtype=jnp.int32)
    masks = {k: (lanes & k) != 0 for k in (8, 4, 2, 1)}
    perm_sub = {k: (lanes - k) % L for k in (8, 4, 2, 1)}
    perm_add = {k: (lanes + k) % L for k in (8, 4, 2, 1)}

    pltpu.async_copy(idx_hbm.at[pl.ds(wrow0, NBUF)], idx_v.at[0], isem.at[0])

    def block_body(t, carry):
        p = lax.rem(t, 2)
        pn = lax.rem(t + 1, 2)
        tn = lax.min(t + 1, N_BLOCKS - 1)
        pltpu.async_copy(
            idx_hbm.at[pl.ds(wrow0 + tn * NBUF, NBUF)], idx_v.at[pn],
            isem.at[pn],
        )
        pltpu.make_async_copy(
            idx_hbm.at[pl.ds(0, NBUF)], idx_v.at[p], isem.at[p]
        ).wait()
        chunk0 = t * NBUF
        for b in range(NBUF):
            pltpu.async_copy(
                lut_hbm.at[idx_v.at[p, b]], rows_v.at[b], gsem.at[b]
            )
        for b in range(NBUF):
            pb = b % 2
            pltpu.make_async_copy(
                lut_hbm.at[idx_v.at[p, b]], rows_v.at[b], gsem.at[b]
            ).wait()

            def _drain_plane():
                pltpu.make_async_copy(
                    plane_v.at[pb], out_hbm.at[0, :, pl.ds(0, CHUNK)],
                    osem.at[pb],
                ).wait()

            if b >= 2:
                _drain_plane()
            else:
                pl.when(t > 0)(_drain_plane)

            def tr_blocks(g, c2):
                b16 = g * L
                for h in range(D_MODEL // L):
                    d0 = h * L
                    m = [
                        rows_v[b, b16 + i, pl.ds(d0, L)] * SCALE
                        for i in range(L)
                    ]
                    for k in (8, 4, 2, 1):
                        for i in range(L):
                            if i & k == 0:
                                j = i | k
                                rr = jnp.take_along_axis(
                                    m[j], perm_sub[k], axis=0
                                )
                                ra = jnp.take_along_axis(
                                    m[i], perm_add[k], axis=0
                                )
                                m[i] = jnp.where(masks[k], rr, m[i])
                                m[j] = jnp.where(masks[k], m[j], ra)
                    for j in range(L):
                        plane_v[pb, d0 + j, pl.ds(b16, L)] = m[j]
                return c2

            lax.fori_loop(0, CHUNK // L, tr_blocks, 0)

            k_flat = wrow0 + chunk0 + b
            s = k_flat // JBLK
            b0 = pl.multiple_of((k_flat % JBLK) * CHUNK, CHUNK)
            pltpu.async_copy(
                plane_v.at[pb], out_hbm.at[s, :, pl.ds(b0, CHUNK)],
                osem.at[pb],
            )
        return carry

    lax.fori_loop(0, N_BLOCKS, block_body, 0)
    for pb in range(2):
        pltpu.make_async_copy(
            plane_v.at[pb], out_hbm.at[0, :, pl.ds(0, CHUNK)], osem.at[pb]
        ).wait()
    pltpu.make_async_copy(
        idx_hbm.at[pl.ds(0, NBUF)], idx_v.at[N_BLOCKS % 2],
        isem.at[N_BLOCKS % 2],
    ).wait()


def kernel(x, lut):
    lut_t = jnp.swapaxes(lut, 0, 1)
    tail = jnp.pad(
        lut[N_RPK * CHUNK :, :], ((0, CHUNK - D_MODEL), (0, PAD_D - D_MODEL))
    )
    table = _repack_sc(lut_t, tail)
    idx = jnp.swapaxes(x, 0, 1).astype(jnp.int32).reshape(N_CHUNKS, CHUNK)
    o_t = _embed_sc(table, idx)
    return o_t.transpose(2, 0, 1)


# speedup vs baseline: 1.6719x; 1.2483x over previous
import functools
import math

import jax
import jax.numpy as jnp
from jax import lax
from jax.experimental import pallas as pl
from jax.experimental.pallas import tpu as pltpu
from jax.experimental.pallas import tpu_sc as plsc

D_MODEL = 64
SCALE = math.sqrt(D_MODEL)
VOCAB = 1000000
PAD_D = 128

_info = plsc.get_sparse_core_info()
NC, NS, L = _info.num_cores, _info.num_subcores, _info.num_lanes
NW = NC * NS

SEQ = 200
BATCH = 4096
CHUNK = 128
JBLK = BATCH // CHUNK
N_CHUNKS = SEQ * JBLK
ROWS_PER_W = N_CHUNKS // NW
NBUF = 4
N_BLOCKS = ROWS_PER_W // NBUF


N_RPK = VOCAB // CHUNK


@functools.partial(
    pl.kernel,
    out_type=jax.ShapeDtypeStruct((VOCAB, PAD_D), jnp.float32),
    mesh=plsc.VectorSubcoreMesh(core_axis_name="c", subcore_axis_name="s"),
    scratch_types=[
        pltpu.VMEM((2, D_MODEL, CHUNK), jnp.float32),
        pltpu.VMEM((2, CHUNK, PAD_D), jnp.float32),
        pltpu.SemaphoreType.DMA((2,)),
        pltpu.SemaphoreType.DMA((2,)),
    ],
    compiler_params=pltpu.CompilerParams(needs_layout_passes=False),
)
def _repack_sc(src_hbm, tail_hbm, dst_hbm, sbuf, obuf, osem, rsem):
    wid = lax.axis_index("s") * NC + lax.axis_index("c")
    nb = jnp.where(wid < N_RPK % NW, N_RPK // NW + 1, N_RPK // NW)

    lanes = jnp.arange(L, dtype=jnp.int32)
    masks = {k: (lanes & k) != 0 for k in (8, 4, 2, 1)}
    perm_sub = {k: (lanes - k) % L for k in (8, 4, 2, 1)}
    perm_add = {k: (lanes + k) % L for k in (8, 4, 2, 1)}

    @pl.when(wid == NW - 1)
    def _tail():
        pltpu.sync_copy(tail_hbm, obuf.at[0])
        pltpu.sync_copy(
            obuf.at[0, pl.ds(0, D_MODEL), :],
            dst_hbm.at[pl.ds(N_RPK * CHUNK, D_MODEL)],
        )

    c_first = pl.multiple_of(wid * CHUNK, CHUNK)
    pltpu.async_copy(
        src_hbm.at[:, pl.ds(c_first, CHUNK)], sbuf.at[0], rsem.at[0]
    )

    def body(i, carry):
        p = lax.rem(i, 2)
        pn = lax.rem(i + 1, 2)
        c0 = pl.multiple_of((wid + i * NW) * CHUNK, CHUNK)
        cn = pl.multiple_of(
            (wid + lax.min(i + 1, nb - 1) * NW) * CHUNK, CHUNK
        )
        pltpu.async_copy(
            src_hbm.at[:, pl.ds(cn, CHUNK)], sbuf.at[pn], rsem.at[pn]
        )
        pltpu.make_async_copy(
            src_hbm.at[:, pl.ds(0, CHUNK)], sbuf.at[p], rsem.at[p]
        ).wait()

        def _drain():
            pltpu.make_async_copy(
                obuf.at[p], dst_hbm.at[pl.ds(0, CHUNK)], osem.at[p]
            ).wait()

        pl.when(i > 1)(_drain)

        def tr_blocks(g, c2):
            c16 = g * L
            for h in range(D_MODEL // L):
                d0 = h * L
                m = [sbuf[p, d0 + i2, pl.ds(c16, L)] for i2 in range(L)]
                for k in (8, 4, 2, 1):
                    for i2 in range(L):
                        if i2 & k == 0:
                            j2 = i2 | k
                            rr = jnp.take_along_axis(m[j2], perm_sub[k], axis=0)
                            ra = jnp.take_along_axis(m[i2], perm_add[k], axis=0)
                            m[i2] = jnp.where(masks[k], rr, m[i2])
                            m[j2] = jnp.where(masks[k], m[j2], ra)
                for j2 in range(L):
                    obuf[p, c16 + j2, pl.ds(d0, L)] = m[j2]
            return c2

        lax.fori_loop(0, CHUNK // L, tr_blocks, 0)
        pltpu.async_copy(obuf.at[p], dst_hbm.at[pl.ds(c0, CHUNK)], osem.at[p])
        return carry

    lax.fori_loop(0, nb, body, 0)
    pltpu.make_async_copy(
        src_hbm.at[:, pl.ds(0, CHUNK)], sbuf.at[lax.rem(nb, 2)],
        rsem.at[lax.rem(nb, 2)],
    ).wait()
    for p in range(2):
        pltpu.make_async_copy(
            obuf.at[p], dst_hbm.at[pl.ds(0, CHUNK)], osem.at[p]
        ).wait()


@functools.partial(
    pl.kernel,
    out_type=jax.ShapeDtypeStruct((SEQ, D_MODEL, BATCH), jnp.float32),
    mesh=plsc.VectorSubcoreMesh(core_axis_name="c", subcore_axis_name="s"),
    scratch_types=[
        pltpu.VMEM((2, NBUF, CHUNK), jnp.int32),
        pltpu.VMEM((NBUF, CHUNK, PAD_D), jnp.float32),
        pltpu.VMEM((2, D_MODEL, CHUNK), jnp.float32),
        pltpu.SemaphoreType.DMA((2,)),
        pltpu.SemaphoreType.DMA((NBUF,)),
        pltpu.SemaphoreType.DMA((2,)),
    ],
    compiler_params=pltpu.CompilerParams(needs_layout_passes=False),
)
def _embed_sc(lut_hbm, idx_hbm, out_hbm, idx_v, rows_v, plane_v, isem, gsem, osem):
    wid = lax.axis_index("s") * NC + lax.axis_index("c")
    wrow0 = wid * ROWS_PER_W

    lanes = jnp.arange(L, dtype=jnp.int32)
    masks = {k: (lanes & k) != 0 for k in (8, 4, 2, 1)}
    perm_sub = {k: (lanes - k) % L for k in (8, 4, 2, 1)}
    perm_add = {k: (lanes + k) % L for k in (8, 4, 2, 1)}

    pltpu.async_copy(idx_hbm.at[pl.ds(wrow0, NBUF)], idx_v.at[0], isem.at[0])
    pltpu.make_async_copy(
        idx_hbm.at[pl.ds(0, NBUF)], idx_v.at[0], isem.at[0]
    ).wait()
    for b in range(NBUF):
        pltpu.async_copy(lut_hbm.at[idx_v.at[0, b]], rows_v.at[b], gsem.at[b])

    def block_body(t, carry):
        p = lax.rem(t, 2)
        pn = lax.rem(t + 1, 2)
        tn = lax.min(t + 1, N_BLOCKS - 1)
        pltpu.async_copy(
            idx_hbm.at[pl.ds(wrow0 + tn * NBUF, NBUF)], idx_v.at[pn],
            isem.at[pn],
        )
        chunk0 = t * NBUF
        for b in range(NBUF):
            pb = b % 2
            pltpu.make_async_copy(
                lut_hbm.at[idx_v.at[p, b]], rows_v.at[b], gsem.at[b]
            ).wait()

            def _drain_plane():
                pltpu.make_async_copy(
                    plane_v.at[pb], out_hbm.at[0, :, pl.ds(0, CHUNK)],
                    osem.at[pb],
                ).wait()

            if b >= 2:
                _drain_plane()
            else:
                pl.when(t > 0)(_drain_plane)

            def tr_blocks(g, c2):
                b16 = g * L
                for h in range(D_MODEL // L):
                    d0 = h * L
                    m = [
                        rows_v[b, b16 + i, pl.ds(d0, L)] * SCALE
                        for i in range(L)
                    ]
                    for k in (8, 4, 2, 1):
                        for i in range(L):
                            if i & k == 0:
                                j = i | k
                                rr = jnp.take_along_axis(
                                    m[j], perm_sub[k], axis=0
                                )
                                ra = jnp.take_along_axis(
                                    m[i], perm_add[k], axis=0
                                )
                                m[i] = jnp.where(masks[k], rr, m[i])
                                m[j] = jnp.where(masks[k], m[j], ra)
                    for j in range(L):
                        plane_v[pb, d0 + j, pl.ds(b16, L)] = m[j]
                return c2

            lax.fori_loop(0, CHUNK // L, tr_blocks, 0)

            k_flat = wrow0 + chunk0 + b
            s = k_flat // JBLK
            b0 = pl.multiple_of((k_flat % JBLK) * CHUNK, CHUNK)
            pltpu.async_copy(
                plane_v.at[pb], out_hbm.at[s, :, pl.ds(b0, CHUNK)],
                osem.at[pb],
            )
            if b == 0:
                pltpu.make_async_copy(
                    idx_hbm.at[pl.ds(0, NBUF)], idx_v.at[pn], isem.at[pn]
                ).wait()
            pltpu.async_copy(
                lut_hbm.at[idx_v.at[pn, b]], rows_v.at[b], gsem.at[b]
            )
        return carry

    lax.fori_loop(0, N_BLOCKS, block_body, 0)
    for b in range(NBUF):
        pltpu.make_async_copy(
            lut_hbm.at[idx_v.at[0, b]], rows_v.at[b], gsem.at[b]
        ).wait()
    for pb in range(2):
        pltpu.make_async_copy(
            plane_v.at[pb], out_hbm.at[0, :, pl.ds(0, CHUNK)], osem.at[pb]
        ).wait()


def kernel(x, lut):
    lut_t = jnp.swapaxes(lut, 0, 1)
    tail = jnp.pad(
        lut[N_RPK * CHUNK :, :], ((0, CHUNK - D_MODEL), (0, PAD_D - D_MODEL))
    )
    table = _repack_sc(lut_t, tail)
    idx = jnp.swapaxes(x, 0, 1).astype(jnp.int32).reshape(N_CHUNKS, CHUNK)
    o_t = _embed_sc(table, idx)
    return o_t.transpose(2, 0, 1)
